# Initial kernel scaffold; baseline (speedup 1.0000x reference)
import jax
import jax.numpy as jnp
from jax.experimental import pallas as pl


def kernel(tokens, W1, b1, W2, b2):
    raise NotImplementedError("write your pallas kernel here")


# speedup vs baseline: 1.8406x; 1.8406x over previous
import functools

import jax
import jax.numpy as jnp
from jax.experimental import pallas as pl
from jax.experimental.pallas import tpu as pltpu

_K = 8


def _gate_block(x_ref, w1_ref, b1_ref, w2_ref, b2_ref, r_ref, idx_ref):
    x = x_ref[...]
    h = jax.lax.dot_general(
        x, w1_ref[...], (((1,), (1,)), ((), ())),
        preferred_element_type=jnp.float32)
    h = jnp.maximum(h + b1_ref[...], 0.0)
    logits = jax.lax.dot_general(
        h, w2_ref[...], (((1,), (1,)), ((), ())),
        preferred_element_type=jnp.float32)
    logits = logits + b2_ref[...]

    m = jnp.max(logits, axis=-1, keepdims=True)
    e = jnp.exp(logits - m)
    probs = e / jnp.sum(e, axis=-1, keepdims=True)

    bm, n_exp = probs.shape
    col = jax.lax.broadcasted_iota(jnp.int32, (bm, n_exp), 1)
    kcol = jax.lax.broadcasted_iota(jnp.int32, (bm, _K), 1)

    p = probs
    mask = jnp.zeros((bm, n_exp), dtype=jnp.bool_)
    idx_out = jnp.zeros((bm, _K), dtype=jnp.int32)
    for j in range(_K):
        mj = jnp.max(p, axis=-1, keepdims=True)
        ij = jnp.min(jnp.where(p == mj, col, n_exp), axis=-1, keepdims=True)
        chosen = col == ij
        mask = jnp.logical_or(mask, chosen)
        p = jnp.where(chosen, -jnp.inf, p)
        idx_out = idx_out + jnp.where(kcol == j, ij, 0)

    r_ref[...] = jnp.where(mask, probs, 0.0)
    idx_ref[...] = idx_out


@functools.partial(jax.jit, static_argnames=())
def kernel(tokens, W1, b1, W2, b2):
    B = tokens.shape[0]
    x = tokens.reshape(B, -1)
    D = x.shape[1]
    H = W1.shape[0]
    E = W2.shape[0]
    BM = 256

    b1r = b1.reshape(1, H)
    b2r = b2.reshape(1, E)

    R, idx = pl.pallas_call(
        _gate_block,
        grid=(B // BM,),
        in_specs=[
            pl.BlockSpec((BM, D), lambda i: (i, 0)),
            pl.BlockSpec((H, D), lambda i: (0, 0)),
            pl.BlockSpec((1, H), lambda i: (0, 0)),
            pl.BlockSpec((E, H), lambda i: (0, 0)),
            pl.BlockSpec((1, E), lambda i: (0, 0)),
        ],
        out_specs=[
            pl.BlockSpec((BM, E), lambda i: (i, 0)),
            pl.BlockSpec((BM, _K), lambda i: (i, 0)),
        ],
        out_shape=[
            jax.ShapeDtypeStruct((B, E), jnp.float32),
            jax.ShapeDtypeStruct((B, _K), jnp.int32),
        ],
        compiler_params=pltpu.CompilerParams(
            dimension_semantics=("arbitrary",),
        ),
    )(x, W1, b1r, W2, b2r)
    return (R, idx)


# speedup vs baseline: 2.0899x; 1.1355x over previous
import functools

import jax
import jax.numpy as jnp
from jax.experimental import pallas as pl
from jax.experimental.pallas import tpu as pltpu

_K = 8


def _gate_block(x_ref, w1_ref, b1_ref, w2_ref, b2_ref, r_ref, idx_ref):
    x = x_ref[...]
    h = jax.lax.dot_general(
        x, w1_ref[...], (((1,), (1,)), ((), ())),
        preferred_element_type=jnp.float32)
    h = jnp.maximum(h + b1_ref[...], 0.0)
    logits = jax.lax.dot_general(
        h, w2_ref[...], (((1,), (1,)), ((), ())),
        preferred_element_type=jnp.float32)
    logits = logits + b2_ref[...]

    m = jnp.max(logits, axis=-1, keepdims=True)
    e = jnp.exp(logits - m)
    probs = e / jnp.sum(e, axis=-1, keepdims=True)

    bm, n_exp = probs.shape
    col = jax.lax.broadcasted_iota(jnp.int32, (bm, n_exp), 1)
    kcol = jax.lax.broadcasted_iota(jnp.int32, (bm, _K), 1)

    p = probs
    mask = jnp.zeros((bm, n_exp), dtype=jnp.bool_)
    idx_out = jnp.zeros((bm, _K), dtype=jnp.int32)
    for j in range(_K):
        mj = jnp.max(p, axis=-1, keepdims=True)
        ij = jnp.min(jnp.where(p == mj, col, n_exp), axis=-1, keepdims=True)
        chosen = col == ij
        mask = jnp.logical_or(mask, chosen)
        p = jnp.where(chosen, -jnp.inf, p)
        idx_out = idx_out + jnp.where(kcol == j, ij, 0)

    r_ref[...] = jnp.where(mask, probs, 0.0)
    idx_ref[...] = idx_out


@functools.partial(jax.jit, static_argnames=())
def kernel(tokens, W1, b1, W2, b2):
    B = tokens.shape[0]
    x = tokens.reshape(B, -1)
    D = x.shape[1]
    H = W1.shape[0]
    E = W2.shape[0]
    BM = 512

    b1r = b1.reshape(1, H)
    b2r = b2.reshape(1, E)

    R, idx = pl.pallas_call(
        _gate_block,
        grid=(B // BM,),
        in_specs=[
            pl.BlockSpec((BM, D), lambda i: (i, 0)),
            pl.BlockSpec((H, D), lambda i: (0, 0)),
            pl.BlockSpec((1, H), lambda i: (0, 0)),
            pl.BlockSpec((E, H), lambda i: (0, 0)),
            pl.BlockSpec((1, E), lambda i: (0, 0)),
        ],
        out_specs=[
            pl.BlockSpec((BM, E), lambda i: (i, 0)),
            pl.BlockSpec((BM, _K), lambda i: (i, 0)),
        ],
        out_shape=[
            jax.ShapeDtypeStruct((B, E), jnp.float32),
            jax.ShapeDtypeStruct((B, _K), jnp.int32),
        ],
        compiler_params=pltpu.CompilerParams(
            dimension_semantics=("arbitrary",),
        ),
    )(x, W1, b1r, W2, b2r)
    return (R, idx)


# speedup vs baseline: 2.1704x; 1.0385x over previous
import functools

import jax
import jax.numpy as jnp
from jax.experimental import pallas as pl
from jax.experimental.pallas import tpu as pltpu

_K = 8


def _gate_block(x_ref, w1_ref, b1_ref, w2_ref, b2_ref, r_ref, idx_ref):
    x = x_ref[...]
    h = jax.lax.dot_general(
        x, w1_ref[...], (((1,), (1,)), ((), ())),
        preferred_element_type=jnp.float32)
    h = jnp.maximum(h + b1_ref[...], 0.0)
    logits = jax.lax.dot_general(
        h, w2_ref[...], (((1,), (1,)), ((), ())),
        preferred_element_type=jnp.float32)
    logits = logits + b2_ref[...]

    m = jnp.max(logits, axis=-1, keepdims=True)
    e = jnp.exp(logits - m)
    probs = e / jnp.sum(e, axis=-1, keepdims=True)

    bm, n_exp = probs.shape
    col = jax.lax.broadcasted_iota(jnp.int32, (bm, n_exp), 1)
    kcol = jax.lax.broadcasted_iota(jnp.int32, (bm, _K), 1)

    p = probs
    mask = jnp.zeros((bm, n_exp), dtype=jnp.bool_)
    idx_out = jnp.zeros((bm, _K), dtype=jnp.int32)
    for j in range(_K):
        mj = jnp.max(p, axis=-1, keepdims=True)
        ij = jnp.min(jnp.where(p == mj, col, n_exp), axis=-1, keepdims=True)
        chosen = col == ij
        mask = jnp.logical_or(mask, chosen)
        p = jnp.where(chosen, -jnp.inf, p)
        idx_out = idx_out + jnp.where(kcol == j, ij, 0)

    r_ref[...] = jnp.where(mask, probs, 0.0)
    idx_ref[...] = idx_out


@functools.partial(jax.jit, static_argnames=())
def kernel(tokens, W1, b1, W2, b2):
    B = tokens.shape[0]
    x = tokens.reshape(B, -1)
    D = x.shape[1]
    H = W1.shape[0]
    E = W2.shape[0]
    BM = 1024

    b1r = b1.reshape(1, H)
    b2r = b2.reshape(1, E)

    R, idx = pl.pallas_call(
        _gate_block,
        grid=(B // BM,),
        in_specs=[
            pl.BlockSpec((BM, D), lambda i: (i, 0)),
            pl.BlockSpec((H, D), lambda i: (0, 0)),
            pl.BlockSpec((1, H), lambda i: (0, 0)),
            pl.BlockSpec((E, H), lambda i: (0, 0)),
            pl.BlockSpec((1, E), lambda i: (0, 0)),
        ],
        out_specs=[
            pl.BlockSpec((BM, E), lambda i: (i, 0)),
            pl.BlockSpec((BM, _K), lambda i: (i, 0)),
        ],
        out_shape=[
            jax.ShapeDtypeStruct((B, E), jnp.float32),
            jax.ShapeDtypeStruct((B, _K), jnp.int32),
        ],
        compiler_params=pltpu.CompilerParams(
            dimension_semantics=("arbitrary",),
        ),
    )(x, W1, b1r, W2, b2r)
    return (R, idx)


# speedup vs baseline: 2.3117x; 1.0651x over previous
import dataclasses
import functools

import jax
import jax.numpy as jnp
from jax import lax
from jax.experimental import pallas as pl
from jax.experimental.pallas import tpu as pltpu
from jax.experimental.pallas import tpu_sc as plsc

_K = 8
_NTILES = 32
_LANES = 16


def _gate_block(x_ref, w1_ref, b1_ref, w2_ref, b2_ref, pt_ref):
    x = x_ref[...]
    h = lax.dot_general(
        x, w1_ref[...], (((1,), (1,)), ((), ())),
        preferred_element_type=jnp.float32)
    h = jnp.maximum(h + b1_ref[...], 0.0)
    logits_t = lax.dot_general(
        w2_ref[...], h, (((1,), (1,)), ((), ())),
        preferred_element_type=jnp.float32)
    logits_t = logits_t + b2_ref[...]
    m = jnp.max(logits_t, axis=0, keepdims=True)
    e = jnp.exp(logits_t - m)
    pt_ref[...] = e / jnp.sum(e, axis=0, keepdims=True)


def _probs_t(x, W1, b1, W2, b2):
    B, D = x.shape
    H = W1.shape[0]
    E = W2.shape[0]
    BM = 1024
    return pl.pallas_call(
        _gate_block,
        grid=(B // BM,),
        in_specs=[
            pl.BlockSpec((BM, D), lambda i: (i, 0)),
            pl.BlockSpec((H, D), lambda i: (0, 0)),
            pl.BlockSpec((1, H), lambda i: (0, 0)),
            pl.BlockSpec((E, H), lambda i: (0, 0)),
            pl.BlockSpec((E, 1), lambda i: (0, 0)),
        ],
        out_specs=pl.BlockSpec((E, BM), lambda i: (0, i)),
        out_shape=jax.ShapeDtypeStruct((E, B), jnp.float32),
        compiler_params=pltpu.CompilerParams(
            dimension_semantics=("arbitrary",),
        ),
    )(x, W1, b1.reshape(1, H), W2, b2.reshape(E, 1))


def _topk_scatter(probs_t):
    E, B = probs_t.shape
    rows_per_tile = B // _NTILES
    groups = rows_per_tile // _LANES
    mesh = plsc.VectorSubcoreMesh(core_axis_name="c", subcore_axis_name="s")

    cp = pltpu.CompilerParams()
    if "needs_layout_passes" in pltpu.CompilerParams.__dataclass_fields__:
        cp = dataclasses.replace(cp, needs_layout_passes=False)

    @functools.partial(
        pl.kernel,
        compiler_params=cp,
        out_type=[
            jax.ShapeDtypeStruct((B, E), jnp.float32),
            jax.ShapeDtypeStruct((B, _K), jnp.int32),
        ],
        mesh=mesh,
        scratch_types=[
            pltpu.VMEM((E, rows_per_tile), jnp.float32),
            pltpu.VMEM((rows_per_tile, E), jnp.float32),
            pltpu.VMEM((rows_per_tile, _K), jnp.int32),
        ],
    )
    def sc_kernel(pt_hbm, r_hbm, idx_hbm, pt_v, r_v, idx_v):
        wid = lax.axis_index("s") * 2 + lax.axis_index("c")
        base = wid * rows_per_tile
        pltpu.sync_copy(pt_hbm.at[:, pl.ds(base, rows_per_tile)], pt_v)

        lane = lax.iota(jnp.int32, _LANES)
        zero_v = jnp.zeros((_LANES,), jnp.float32)

        @pl.loop(0, rows_per_tile)
        def _(r):
            @pl.loop(0, E, step=_LANES)
            def _(c):
                r_v[r, pl.ds(c, _LANES)] = zero_v

        @pl.loop(0, groups)
        def _(g):
            row_vec = g * _LANES + lane
            neg = jnp.full((_LANES,), -1.0, jnp.float32)
            zi = jnp.zeros((_LANES,), jnp.int32)
            carry0 = (neg,) * _K + (zi,) * _K

            def body(e, carry):
                t = list(carry[:_K])
                j = list(carry[_K:])
                cur_v = pt_v[e, pl.ds(g * _LANES, _LANES)]
                cur_j = jnp.full((_LANES,), e, jnp.int32)
                for k in range(_K):
                    gt = cur_v > t[k]
                    nt = jnp.where(gt, cur_v, t[k])
                    nj = jnp.where(gt, cur_j, j[k])
                    cur_v = jnp.where(gt, t[k], cur_v)
                    cur_j = jnp.where(gt, j[k], cur_j)
                    t[k] = nt
                    j[k] = nj
                return tuple(t) + tuple(j)

            carry = lax.fori_loop(0, E, body, carry0)
            t = carry[:_K]
            j = carry[_K:]
            for k in range(_K):
                plsc.store_scatter(r_v, [row_vec, j[k]], t[k])
                plsc.store_scatter(
                    idx_v, [row_vec, jnp.full((_LANES,), k, jnp.int32)], j[k])

        pltpu.sync_copy(r_v, r_hbm.at[pl.ds(base, rows_per_tile), :])
        pltpu.sync_copy(idx_v, idx_hbm.at[pl.ds(base, rows_per_tile), :])

    return sc_kernel(probs_t)


def kernel(tokens, W1, b1, W2, b2):
    B = tokens.shape[0]
    x = tokens.reshape(B, -1)
    probs_t = _probs_t(x, W1, b1, W2, b2)
    R, idx = _topk_scatter(probs_t)
    return (R, idx)


# speedup vs baseline: 2.3147x; 1.0013x over previous
import dataclasses
import functools

import jax
import jax.numpy as jnp
from jax import lax
from jax.experimental import pallas as pl
from jax.experimental.pallas import tpu as pltpu
from jax.experimental.pallas import tpu_sc as plsc

_K = 8
_NTILES = 32
_LANES = 16


def _gate_block(x_ref, w1_ref, b1_ref, w2_ref, b2_ref, pt_ref):
    x = x_ref[...]
    h = lax.dot_general(
        x, w1_ref[...], (((1,), (1,)), ((), ())),
        preferred_element_type=jnp.float32)
    h = jnp.maximum(h + b1_ref[...], 0.0)
    logits_t = lax.dot_general(
        w2_ref[...], h, (((1,), (1,)), ((), ())),
        preferred_element_type=jnp.float32)
    logits_t = logits_t + b2_ref[...]
    m = jnp.max(logits_t, axis=0, keepdims=True)
    e = jnp.exp(logits_t - m)
    pt_ref[...] = e / jnp.sum(e, axis=0, keepdims=True)


def _probs_t(x, W1, b1, W2, b2):
    B, D = x.shape
    H = W1.shape[0]
    E = W2.shape[0]
    BM = 1024
    return pl.pallas_call(
        _gate_block,
        grid=(B // BM,),
        in_specs=[
            pl.BlockSpec((BM, D), lambda i: (i, 0)),
            pl.BlockSpec((H, D), lambda i: (0, 0)),
            pl.BlockSpec((1, H), lambda i: (0, 0)),
            pl.BlockSpec((E, H), lambda i: (0, 0)),
            pl.BlockSpec((E, 1), lambda i: (0, 0)),
        ],
        out_specs=pl.BlockSpec((E, BM), lambda i: (0, i)),
        out_shape=jax.ShapeDtypeStruct((E, B), jnp.float32),
        compiler_params=pltpu.CompilerParams(
            dimension_semantics=("arbitrary",),
        ),
    )(x, W1, b1.reshape(1, H), W2, b2.reshape(E, 1))


def _topk_scatter(probs_t):
    E, B = probs_t.shape
    rows_per_tile = B // _NTILES
    groups = rows_per_tile // _LANES
    mesh = plsc.VectorSubcoreMesh(core_axis_name="c", subcore_axis_name="s")

    cp = pltpu.CompilerParams()
    if "needs_layout_passes" in pltpu.CompilerParams.__dataclass_fields__:
        cp = dataclasses.replace(cp, needs_layout_passes=False)
    if "use_tc_tiling_on_sc" in pltpu.CompilerParams.__dataclass_fields__:
        cp = dataclasses.replace(cp, use_tc_tiling_on_sc=True)

    @functools.partial(
        pl.kernel,
        compiler_params=cp,
        out_type=[
            jax.ShapeDtypeStruct((B, E), jnp.float32),
            jax.ShapeDtypeStruct((B, _K), jnp.int32),
        ],
        mesh=mesh,
        scratch_types=[
            pltpu.VMEM((E, rows_per_tile), jnp.float32),
            pltpu.VMEM((rows_per_tile, E), jnp.float32),
            pltpu.VMEM((rows_per_tile, _K), jnp.int32),
        ],
    )
    def sc_kernel(pt_hbm, r_hbm, idx_hbm, pt_v, r_v, idx_v):
        wid = lax.axis_index("s") * 2 + lax.axis_index("c")
        base = wid * rows_per_tile
        pltpu.sync_copy(pt_hbm.at[:, pl.ds(base, rows_per_tile)], pt_v)

        lane = lax.iota(jnp.int32, _LANES)
        zero_v = jnp.zeros((_LANES,), jnp.float32)

        @pl.loop(0, rows_per_tile)
        def _(r):
            @pl.loop(0, E, step=_LANES)
            def _(c):
                r_v[r, pl.ds(c, _LANES)] = zero_v

        @pl.loop(0, groups)
        def _(g):
            row_vec = g * _LANES + lane
            neg = jnp.full((_LANES,), -1.0, jnp.float32)
            zi = jnp.zeros((_LANES,), jnp.int32)
            carry0 = (neg,) * _K + (zi,) * _K

            def body(e, carry):
                t = list(carry[:_K])
                j = list(carry[_K:])
                cur_v = pt_v[e, pl.ds(g * _LANES, _LANES)]
                cur_j = jnp.full((_LANES,), e, jnp.int32)
                for k in range(_K):
                    gt = cur_v > t[k]
                    nt = jnp.where(gt, cur_v, t[k])
                    nj = jnp.where(gt, cur_j, j[k])
                    cur_v = jnp.where(gt, t[k], cur_v)
                    cur_j = jnp.where(gt, j[k], cur_j)
                    t[k] = nt
                    j[k] = nj
                return tuple(t) + tuple(j)

            carry = lax.fori_loop(0, E, body, carry0)
            t = carry[:_K]
            j = carry[_K:]
            for k in range(_K):
                plsc.store_scatter(r_v, [row_vec, j[k]], t[k])
                plsc.store_scatter(
                    idx_v, [row_vec, jnp.full((_LANES,), k, jnp.int32)], j[k])

        pltpu.sync_copy(r_v, r_hbm.at[pl.ds(base, rows_per_tile), :])
        pltpu.sync_copy(idx_v, idx_hbm.at[pl.ds(base, rows_per_tile), :])

    return sc_kernel(probs_t)


def kernel(tokens, W1, b1, W2, b2):
    B = tokens.shape[0]
    x = tokens.reshape(B, -1)
    probs_t = _probs_t(x, W1, b1, W2, b2)
    R, idx = _topk_scatter(probs_t)
    return (R, idx)


# speedup vs baseline: 3.1114x; 1.3442x over previous
import dataclasses
import functools

import jax
import jax.numpy as jnp
from jax import lax
from jax.experimental import pallas as pl
from jax.experimental.pallas import tpu as pltpu
from jax.experimental.pallas import tpu_sc as plsc

_K = 8
_NTILES = 32
_LANES = 16


def _gate_block(x_ref, w1_ref, b1_ref, w2_ref, b2_ref, pt_ref):
    bm, C, DC = x_ref.shape
    h = None
    for c in range(C):
        xc = x_ref[:, c, :]
        w1c = w1_ref[:, pl.ds(c * DC, DC)]
        part = lax.dot_general(
            xc, w1c, (((1,), (1,)), ((), ())),
            preferred_element_type=jnp.float32)
        h = part if h is None else h + part
    h = jnp.maximum(h + b1_ref[...], 0.0)
    logits_t = lax.dot_general(
        w2_ref[...], h, (((1,), (1,)), ((), ())),
        preferred_element_type=jnp.float32)
    logits_t = logits_t + b2_ref[...]
    m = jnp.max(logits_t, axis=0, keepdims=True)
    e = jnp.exp(logits_t - m)
    pt_ref[...] = e / jnp.sum(e, axis=0, keepdims=True)


def _probs_t(tokens, W1, b1, W2, b2):
    B, C, DC = tokens.shape
    H = W1.shape[0]
    E = W2.shape[0]
    BM = 256
    return pl.pallas_call(
        _gate_block,
        grid=(B // BM,),
        in_specs=[
            pl.BlockSpec((BM, C, DC), lambda i: (i, 0, 0)),
            pl.BlockSpec((H, C * DC), lambda i: (0, 0)),
            pl.BlockSpec((1, H), lambda i: (0, 0)),
            pl.BlockSpec((E, H), lambda i: (0, 0)),
            pl.BlockSpec((E, 1), lambda i: (0, 0)),
        ],
        out_specs=pl.BlockSpec((E, BM), lambda i: (0, i)),
        out_shape=jax.ShapeDtypeStruct((E, B), jnp.float32),
        compiler_params=pltpu.CompilerParams(
            dimension_semantics=("arbitrary",),
        ),
    )(tokens, W1, b1.reshape(1, H), W2, b2.reshape(E, 1))


def _topk_scatter(probs_t):
    E, B = probs_t.shape
    rows_per_tile = B // _NTILES
    groups = rows_per_tile // _LANES
    mesh = plsc.VectorSubcoreMesh(core_axis_name="c", subcore_axis_name="s")

    cp = pltpu.CompilerParams()
    if "needs_layout_passes" in pltpu.CompilerParams.__dataclass_fields__:
        cp = dataclasses.replace(cp, needs_layout_passes=False)
    if "use_tc_tiling_on_sc" in pltpu.CompilerParams.__dataclass_fields__:
        cp = dataclasses.replace(cp, use_tc_tiling_on_sc=True)

    @functools.partial(
        pl.kernel,
        compiler_params=cp,
        out_type=[
            jax.ShapeDtypeStruct((B, E), jnp.float32),
            jax.ShapeDtypeStruct((B, _K), jnp.int32),
        ],
        mesh=mesh,
        scratch_types=[
            pltpu.VMEM((E, rows_per_tile), jnp.float32),
            pltpu.VMEM((rows_per_tile, E), jnp.float32),
            pltpu.VMEM((rows_per_tile, _K), jnp.int32),
        ],
    )
    def sc_kernel(pt_hbm, r_hbm, idx_hbm, pt_v, r_v, idx_v):
        wid = lax.axis_index("s") * 2 + lax.axis_index("c")
        base = wid * rows_per_tile
        pltpu.sync_copy(pt_hbm.at[:, pl.ds(base, rows_per_tile)], pt_v)

        lane = lax.iota(jnp.int32, _LANES)
        zero_v = jnp.zeros((_LANES,), jnp.float32)

        @pl.loop(0, rows_per_tile)
        def _(r):
            @pl.loop(0, E, step=_LANES)
            def _(c):
                r_v[r, pl.ds(c, _LANES)] = zero_v

        @pl.loop(0, groups)
        def _(g):
            row_vec = g * _LANES + lane
            neg = jnp.full((_LANES,), -1.0, jnp.float32)
            zi = jnp.zeros((_LANES,), jnp.int32)
            carry0 = (neg,) * _K + (zi,) * _K

            def body(e, carry):
                t = list(carry[:_K])
                j = list(carry[_K:])
                cur_v = pt_v[e, pl.ds(g * _LANES, _LANES)]
                cur_j = jnp.full((_LANES,), e, jnp.int32)
                for k in range(_K):
                    gt = cur_v > t[k]
                    nt = jnp.where(gt, cur_v, t[k])
                    nj = jnp.where(gt, cur_j, j[k])
                    cur_v = jnp.where(gt, t[k], cur_v)
                    cur_j = jnp.where(gt, j[k], cur_j)
                    t[k] = nt
                    j[k] = nj
                return tuple(t) + tuple(j)

            carry = lax.fori_loop(0, E, body, carry0)
            t = carry[:_K]
            j = carry[_K:]
            for k in range(_K):
                plsc.store_scatter(r_v, [row_vec, j[k]], t[k])
                plsc.store_scatter(
                    idx_v, [row_vec, jnp.full((_LANES,), k, jnp.int32)], j[k])

        pltpu.sync_copy(r_v, r_hbm.at[pl.ds(base, rows_per_tile), :])
        pltpu.sync_copy(idx_v, idx_hbm.at[pl.ds(base, rows_per_tile), :])

    return sc_kernel(probs_t)


def kernel(tokens, W1, b1, W2, b2):
    probs_t = _probs_t(tokens, W1, b1, W2, b2)
    R, idx = _topk_scatter(probs_t)
    return (R, idx)


# speedup vs baseline: 3.4215x; 1.0997x over previous
import dataclasses
import functools

import jax
import jax.numpy as jnp
from jax import lax
from jax.experimental import pallas as pl
from jax.experimental.pallas import tpu as pltpu
from jax.experimental.pallas import tpu_sc as plsc

_K = 8
_NTILES = 32
_LANES = 16


def _gate_block(x_ref, w1_ref, b1_ref, w2_ref, b2_ref, pt_ref):
    bm, C, DC = x_ref.shape
    h = None
    for c in range(C):
        xc = x_ref[:, c, :]
        w1c = w1_ref[:, pl.ds(c * DC, DC)]
        part = lax.dot_general(
            xc, w1c, (((1,), (1,)), ((), ())),
            preferred_element_type=jnp.float32)
        h = part if h is None else h + part
    h = jnp.maximum(h + b1_ref[...], 0.0)
    logits_t = lax.dot_general(
        w2_ref[...], h, (((1,), (1,)), ((), ())),
        preferred_element_type=jnp.float32)
    logits_t = logits_t + b2_ref[...]
    m = jnp.max(logits_t, axis=0, keepdims=True)
    e = jnp.exp(logits_t - m)
    pt_ref[...] = e / jnp.sum(e, axis=0, keepdims=True)


def _probs_t(tokens, W1, b1, W2, b2):
    B, C, DC = tokens.shape
    H = W1.shape[0]
    E = W2.shape[0]
    BM = 512
    return pl.pallas_call(
        _gate_block,
        grid=(B // BM,),
        in_specs=[
            pl.BlockSpec((BM, C, DC), lambda i: (i, 0, 0)),
            pl.BlockSpec((H, C * DC), lambda i: (0, 0)),
            pl.BlockSpec((1, H), lambda i: (0, 0)),
            pl.BlockSpec((E, H), lambda i: (0, 0)),
            pl.BlockSpec((E, 1), lambda i: (0, 0)),
        ],
        out_specs=pl.BlockSpec((E, BM), lambda i: (0, i)),
        out_shape=jax.ShapeDtypeStruct((E, B), jnp.float32),
        compiler_params=pltpu.CompilerParams(
            dimension_semantics=("arbitrary",),
        ),
    )(tokens, W1, b1.reshape(1, H), W2, b2.reshape(E, 1))


def _topk_scatter(probs_t):
    E, B = probs_t.shape
    rows_per_tile = B // _NTILES
    groups = rows_per_tile // _LANES
    mesh = plsc.VectorSubcoreMesh(core_axis_name="c", subcore_axis_name="s")

    cp = pltpu.CompilerParams()
    if "needs_layout_passes" in pltpu.CompilerParams.__dataclass_fields__:
        cp = dataclasses.replace(cp, needs_layout_passes=False)
    if "use_tc_tiling_on_sc" in pltpu.CompilerParams.__dataclass_fields__:
        cp = dataclasses.replace(cp, use_tc_tiling_on_sc=True)

    @functools.partial(
        pl.kernel,
        compiler_params=cp,
        out_type=[
            jax.ShapeDtypeStruct((B, E), jnp.float32),
            jax.ShapeDtypeStruct((B, _K), jnp.int32),
        ],
        mesh=mesh,
        scratch_types=[
            pltpu.VMEM((E, rows_per_tile), jnp.float32),
            pltpu.VMEM((rows_per_tile, E), jnp.float32),
            pltpu.VMEM((rows_per_tile, _K), jnp.int32),
        ],
    )
    def sc_kernel(pt_hbm, r_hbm, idx_hbm, pt_v, r_v, idx_v):
        wid = lax.axis_index("s") * 2 + lax.axis_index("c")
        base = wid * rows_per_tile
        pltpu.sync_copy(pt_hbm.at[:, pl.ds(base, rows_per_tile)], pt_v)

        lane = lax.iota(jnp.int32, _LANES)
        zero_v = jnp.zeros((_LANES,), jnp.float32)

        @pl.loop(0, rows_per_tile)
        def _(r):
            @pl.loop(0, E, step=_LANES)
            def _(c):
                r_v[r, pl.ds(c, _LANES)] = zero_v

        @pl.loop(0, groups)
        def _(g):
            row_vec = g * _LANES + lane
            neg = jnp.full((_LANES,), -1.0, jnp.float32)
            zi = jnp.zeros((_LANES,), jnp.int32)
            carry0 = (neg,) * _K + (zi,) * _K

            def body(e, carry):
                t = list(carry[:_K])
                j = list(carry[_K:])
                cur_v = pt_v[e, pl.ds(g * _LANES, _LANES)]
                cur_j = jnp.full((_LANES,), e, jnp.int32)
                for k in range(_K):
                    gt = cur_v > t[k]
                    nt = jnp.where(gt, cur_v, t[k])
                    nj = jnp.where(gt, cur_j, j[k])
                    cur_v = jnp.where(gt, t[k], cur_v)
                    cur_j = jnp.where(gt, j[k], cur_j)
                    t[k] = nt
                    j[k] = nj
                return tuple(t) + tuple(j)

            carry = lax.fori_loop(0, E, body, carry0)
            t = carry[:_K]
            j = carry[_K:]
            for k in range(_K):
                plsc.store_scatter(r_v, [row_vec, j[k]], t[k])
                plsc.store_scatter(
                    idx_v, [row_vec, jnp.full((_LANES,), k, jnp.int32)], j[k])

        pltpu.sync_copy(r_v, r_hbm.at[pl.ds(base, rows_per_tile), :])
        pltpu.sync_copy(idx_v, idx_hbm.at[pl.ds(base, rows_per_tile), :])

    return sc_kernel(probs_t)


def kernel(tokens, W1, b1, W2, b2):
    probs_t = _probs_t(tokens, W1, b1, W2, b2)
    R, idx = _topk_scatter(probs_t)
    return (R, idx)
